# Initial kernel scaffold; baseline (speedup 1.0000x reference)
import jax
import jax.numpy as jnp
from jax.experimental import pallas as pl


def kernel(x, edge_index, W1, b1, W2, b2, W3, b3, Wt, Ws1, bs1, Ws2, bs2, Wh1, bh1, Wh2, bh2):
    raise NotImplementedError("write your pallas kernel here")


# speedup vs baseline: 18.1437x; 18.1437x over previous
import functools

import jax
import jax.numpy as jnp
from jax import lax
from jax.experimental import pallas as pl
from jax.experimental.pallas import tpu as pltpu
from jax.experimental.pallas import tpu_sc as plsc

N = 10000
E = 320000
H = 128
C_HID = 64
TAU0 = 1.0

NPAD = 10240
BN = 1024
GRID = NPAD // BN
NC = 2
NS = 16
NW = NC * NS
B = 128
NB = 79
PER_W = NB * B
EPAD = NW * PER_W
RPT = NPAD // NS

_f32 = jnp.float32
_i32 = jnp.int32


def _mesh():
    return plsc.VectorSubcoreMesh(core_axis_name="c", subcore_axis_name="s")


def _make_degdc():
    @functools.partial(
        pl.kernel, mesh=_mesh(),
        out_type=[jax.ShapeDtypeStruct((NC, NPAD), _f32),
                  jax.ShapeDtypeStruct((NC, NPAD), _f32)],
        scratch_types=[
            pltpu.VMEM((NB, B), _i32),
            pltpu.VMEM((NB, B), _i32),
            pltpu.VMEM((B,), _f32),
            pltpu.VMEM((B,), _f32),
            pltpu.VMEM_SHARED((NPAD,), _f32),
            pltpu.VMEM_SHARED((NPAD,), _f32),
        ],
    )
    def degdc(rowp2, colp2, zeros1, degp_out, dcp_out,
              row_v, col_v, ones_v, mask_v, deg_sh, dc_sh):
        c = lax.axis_index("c")
        s = lax.axis_index("s")
        wid = s * NC + c
        pltpu.sync_copy(zeros1, deg_sh.at[pl.ds(s * RPT, RPT)])
        pltpu.sync_copy(zeros1, dc_sh.at[pl.ds(s * RPT, RPT)])
        pltpu.sync_copy(rowp2.at[wid], row_v)
        pltpu.sync_copy(colp2.at[wid], col_v)
        for k in range(B // 16):
            ones_v[pl.ds(k * 16, 16)] = jnp.full((16,), 1.0, _f32)
        plsc.subcore_barrier()

        def batch(b, carry):
            def lanes(k, carry2):
                rr = row_v[b, pl.ds(k * 16, 16)]
                cc = col_v[b, pl.ds(k * 16, 16)]
                mask_v[pl.ds(k * 16, 16)] = jnp.where(
                    rr != cc, jnp.full((16,), 1.0, _f32),
                    jnp.zeros((16,), _f32))
                return carry2
            lax.fori_loop(0, B // 16, lanes, 0)
            pltpu.sync_copy(ones_v, deg_sh.at[col_v.at[b]], add=True)
            pltpu.sync_copy(mask_v, dc_sh.at[row_v.at[b]], add=True)
            return carry
        lax.fori_loop(0, NB, batch, 0)

        plsc.subcore_barrier()
        pltpu.sync_copy(deg_sh.at[pl.ds(s * RPT, RPT)],
                        degp_out.at[c, pl.ds(s * RPT, RPT)])
        pltpu.sync_copy(dc_sh.at[pl.ds(s * RPT, RPT)],
                        dcp_out.at[c, pl.ds(s * RPT, RPT)])
    return degdc


def _make_edge_pass(do_agg, do_rs):
    out_type = []
    if do_agg:
        out_type.append(jax.ShapeDtypeStruct((NC, NPAD, H), _f32))
    if do_rs:
        out_type.append(jax.ShapeDtypeStruct((NC, NPAD), _f32))
    scratch = [pltpu.VMEM((NB, B), _i32), pltpu.VMEM((NB, B), _i32)]
    if do_agg:
        scratch += [pltpu.VMEM((B, H), _f32),
                    pltpu.VMEM_SHARED((NPAD, H), _f32),
                    pltpu.SemaphoreType.DMA]
    if do_rs:
        scratch += [pltpu.VMEM((B,), _f32),
                    pltpu.VMEM((B,), _f32),
                    pltpu.VMEM_SHARED((NPAD,), _f32),
                    pltpu.SemaphoreType.DMA]

    @functools.partial(pl.kernel, mesh=_mesh(), out_type=out_type,
                       scratch_types=scratch)
    def edge_pass(*refs):
        it = iter(refs)
        hp = next(it) if do_agg else None
        rowp2 = next(it)
        colp2 = next(it)
        zeros2d = next(it) if do_agg else None
        soft = next(it) if do_rs else None
        zeros1 = next(it) if do_rs else None
        aggp_out = next(it) if do_agg else None
        rsp_out = next(it) if do_rs else None
        row_v = next(it)
        col_v = next(it)
        if do_agg:
            vals_v = next(it)
            agg_sh = next(it)
            sem = next(it)
        if do_rs:
            nse_v = next(it)
            sval_v = next(it)
            rs_sh = next(it)
            sem_rs = next(it)

        c = lax.axis_index("c")
        s = lax.axis_index("s")
        wid = s * NC + c
        if do_agg:
            for j in range(RPT // B):
                pltpu.sync_copy(zeros2d,
                                agg_sh.at[pl.ds(s * RPT + j * B, B)])
        if do_rs:
            pltpu.sync_copy(zeros1, rs_sh.at[pl.ds(s * RPT, RPT)])
        pltpu.sync_copy(rowp2.at[wid], row_v)
        pltpu.sync_copy(colp2.at[wid], col_v)
        plsc.subcore_barrier()

        def batch(b, carry):
            if do_agg:
                pltpu.async_copy(hp.at[col_v.at[b]], vals_v, sem).wait()
                pltpu.sync_copy(vals_v, agg_sh.at[row_v.at[b]], add=True)
            if do_rs:
                pltpu.async_copy(soft.at[col_v.at[b]], sval_v,
                                 sem_rs).wait()

                def lanes(k, carry2):
                    rr = row_v[b, pl.ds(k * 16, 16)]
                    cc = col_v[b, pl.ds(k * 16, 16)]
                    sv = sval_v[pl.ds(k * 16, 16)]
                    nse_v[pl.ds(k * 16, 16)] = jnp.where(
                        rr != cc, sv, jnp.zeros((16,), _f32))
                    return carry2
                lax.fori_loop(0, B // 16, lanes, 0)
                pltpu.sync_copy(nse_v, rs_sh.at[row_v.at[b]], add=True)
            return carry
        lax.fori_loop(0, NB, batch, 0)

        plsc.subcore_barrier()
        if do_agg:
            pltpu.sync_copy(agg_sh.at[pl.ds(s * RPT, RPT)],
                            aggp_out.at[c, pl.ds(s * RPT, RPT)])
        if do_rs:
            pltpu.sync_copy(rs_sh.at[pl.ds(s * RPT, RPT)],
                            rsp_out.at[c, pl.ds(s * RPT, RPT)])
    return edge_pass


_degdc_kernel = _make_degdc()
_agg_kernel = _make_edge_pass(True, False)
_agg_rs_kernel = _make_edge_pass(True, True)
_rs_kernel = _make_edge_pass(False, True)


def _full(shape):
    return pl.BlockSpec(shape, lambda i: tuple(0 for _ in shape))


def _rows(width):
    return pl.BlockSpec((BN, width), lambda i: (i, 0))


def _dot(a, b):
    return jnp.dot(a, b, preferred_element_type=_f32)


_SQRT_HALF = 0.7071067811865476


def _gelu(x):
    return 0.5 * x * (1.0 - lax.erf(-x * _SQRT_HALF))


def _temp_soft(xb, wt_t, ws1, bs1, ws2_t, bs2, g, soft_prev):
    xw = jnp.sum(xb * wt_t, axis=1, keepdims=True)
    val = jax.nn.softplus(xw) + TAU0
    temp = 1.0 / val
    temp = jnp.where(jnp.isinf(temp), 0.0, temp)
    a = jax.nn.relu(_dot(xb, ws1) + bs1)
    l0 = jnp.sum(a * ws2_t[0:1, :], axis=1, keepdims=True) + bs2[0:1, 0:1]
    l1 = jnp.sum(a * ws2_t[1:2, :], axis=1, keepdims=True) + bs2[0:1, 1:2]
    a0 = (l0 + g[:, 0:1]) / temp
    a1 = (l1 + g[:, 1:2]) / temp
    m = jnp.maximum(a0, a1)
    e0 = jnp.exp(a0 - m)
    e1 = jnp.exp(a1 - m)
    den = e0 + e1
    sdf = ((e1 / den) > (e0 / den)).astype(_f32)
    if soft_prev is None:
        return temp, sdf
    return temp, soft_prev + sdf * (1.0 - soft_prev)


def _hard_update(xb, rs0, rs1, dcc, wh1a, wh1b, bh1, wh2_t, bh2, g,
                 temp, softb, z, hard, exitl, li):
    readiness = (rs0 + rs1) / dcc
    a = jax.nn.relu(_dot(xb, wh1a) + readiness * wh1b + bh1)
    l0 = jnp.sum(a * wh2_t[0:1, :], axis=1, keepdims=True) + bh2[0:1, 0:1]
    l1 = jnp.sum(a * wh2_t[1:2, :], axis=1, keepdims=True) + bh2[0:1, 1:2]
    eff0 = jnp.where(softb > 0.0, l0, jnp.float32(1000000.0))
    eff1 = jnp.where(softb > 0.0, l1, jnp.float32(0.0))
    a0 = (eff0 + g[:, 0:1]) / temp
    a1 = (eff1 + g[:, 1:2]) / temp
    m = jnp.maximum(a0, a1)
    e0 = jnp.exp(a0 - m)
    e1 = jnp.exp(a1 - m)
    den = e0 + e1
    nh = ((e1 / den) > (e0 / den)).astype(_f32) * (1.0 - hard)
    z_new = z + xb * nh
    exit_new = jnp.where(nh > 0.0, jnp.full_like(exitl, li), exitl)
    hard_new = hard + nh
    return z_new, exit_new, hard_new, nh


def _tc_prep(x_pad, W1, deg0, deg1, dc0, dc1):
    def body(x_ref, w_ref, d0_ref, d1_ref, c0_ref, c1_ref,
             hp_ref, dinv_ref, dcc_ref):
        deg = d0_ref[...] + d1_ref[...] + 1.0
        dinv = lax.rsqrt(deg)
        dcc_ref[...] = jnp.maximum(c0_ref[...] + c1_ref[...], 1.0)
        h = _dot(x_ref[...], w_ref[...])
        hp_ref[...] = h * dinv
        dinv_ref[...] = dinv
    return pl.pallas_call(
        body, grid=(GRID,),
        in_specs=[_rows(H), _full((H, H)), _rows(1), _rows(1), _rows(1),
                  _rows(1)],
        out_specs=[_rows(H), _rows(1), _rows(1)],
        out_shape=[jax.ShapeDtypeStruct((NPAD, H), _f32),
                   jax.ShapeDtypeStruct((NPAD, 1), _f32),
                   jax.ShapeDtypeStruct((NPAD, 1), _f32)],
    )(x_pad, W1, deg0, deg1, dc0, dc1)


def _tc_layer0(aggp0, aggp1, hp0, dinv, b1r, wt_t, ws1, bs1r, ws2_t, bs2r,
               gs0, W2):
    def body(p0_ref, p1_ref, hp_ref, dinv_ref, b_ref, wt_ref, ws1_ref,
             bs1_ref, ws2_ref, bs2_ref, g_ref, wn_ref,
             x_ref, temp_ref, soft_ref, hpn_ref):
        dinv = dinv_ref[...]
        agg = dinv * (p0_ref[...] + p1_ref[...]) + dinv * hp_ref[...] \
            + b_ref[...]
        x1 = _gelu(agg)
        temp, soft = _temp_soft(x1, wt_ref[...], ws1_ref[...], bs1_ref[...],
                                ws2_ref[...], bs2_ref[...], g_ref[...], None)
        x_ref[...] = x1
        temp_ref[...] = temp
        soft_ref[...] = soft
        hpn_ref[...] = _dot(x1, wn_ref[...]) * dinv
    return pl.pallas_call(
        body, grid=(GRID,),
        in_specs=[_rows(H), _rows(H), _rows(H), _rows(1), _full((1, H)),
                  _full((1, H)), _full((H, C_HID)), _full((1, C_HID)),
                  _full((2, C_HID)), _full((1, 2)), _rows(2),
                  _full((H, H))],
        out_specs=[_rows(H), _rows(1), _rows(1), _rows(H)],
        out_shape=[jax.ShapeDtypeStruct((NPAD, H), _f32),
                   jax.ShapeDtypeStruct((NPAD, 1), _f32),
                   jax.ShapeDtypeStruct((NPAD, 1), _f32),
                   jax.ShapeDtypeStruct((NPAD, H), _f32)],
    )(aggp0, aggp1, hp0, dinv, b1r, wt_t, ws1, bs1r, ws2_t, bs2r, gs0, W2)


def _tc_mid(li_hard, act, has_next):
    def call(aggp0, aggp1, hp_prev, dinv, b_l, x_prev, temp_prev, soft_cur,
             rs0, rs1, dcc, z, hard, exitl, gh, wt_t, ws1, bs1r, ws2_t,
             bs2r, wh1a, wh1b, bh1r, wh2_t, bh2r, gs, *wn):
        def body(*refs):
            (p0_ref, p1_ref, hp_ref, dinv_ref, b_ref, xp_ref, tp_ref,
             sc_ref, r0_ref, r1_ref, dcc_ref, z_ref, hd_ref, ex_ref,
             gh_ref, wt_ref, ws1_ref, bs1_ref, ws2_ref, bs2_ref, wh1a_ref,
             wh1b_ref, bh1_ref, wh2_ref, bh2_ref, gs_ref) = refs[:26]
            idx = 26
            wn_ref = refs[idx] if has_next else None
            idx += 1 if has_next else 0
            outs = refs[idx:]
            (x_ref, temp_ref, soft_ref, z_out, ex_out, hd_out,
             cnt_ref) = outs[:7]
            hpn_ref = outs[7] if has_next else None

            i = pl.program_id(0)
            z_new, exit_new, hard_new, nh = _hard_update(
                xp_ref[...], r0_ref[...], r1_ref[...], dcc_ref[...],
                wh1a_ref[...], wh1b_ref[...], bh1_ref[...], wh2_ref[...],
                bh2_ref[...], gh_ref[...], tp_ref[...], sc_ref[...],
                z_ref[...], hd_ref[...], ex_ref[...], li_hard)
            z_out[...] = z_new
            ex_out[...] = exit_new
            hd_out[...] = hard_new
            ids = lax.broadcasted_iota(_i32, (BN, 1), 0) + i * BN
            cnt = jnp.sum(jnp.where(ids < N, nh, 0.0)).astype(_i32)
            prev = cnt_ref[...]
            base = jnp.where(i == 0, jnp.zeros_like(prev), prev)
            cnt_ref[...] = base + cnt[None, None]

            dinv = dinv_ref[...]
            agg = dinv * (p0_ref[...] + p1_ref[...]) + dinv * hp_ref[...] \
                + b_ref[...]
            x_new = _gelu(agg) if act else agg
            temp, soft = _temp_soft(
                x_new, wt_ref[...], ws1_ref[...], bs1_ref[...], ws2_ref[...],
                bs2_ref[...], gs_ref[...], sc_ref[...])
            x_ref[...] = x_new
            temp_ref[...] = temp
            soft_ref[...] = soft
            if has_next:
                hpn_ref[...] = _dot(x_new, wn_ref[...]) * dinv

        in_specs = [_rows(H), _rows(H), _rows(H), _rows(1), _full((1, H)),
                    _rows(H), _rows(1), _rows(1), _rows(1), _rows(1),
                    _rows(1), _rows(H), _rows(1), _rows(1), _rows(2),
                    _full((1, H)), _full((H, C_HID)), _full((1, C_HID)),
                    _full((2, C_HID)), _full((1, 2)), _full((H, C_HID)),
                    _full((1, C_HID)), _full((1, C_HID)),
                    _full((2, C_HID)), _full((1, 2)), _rows(2)]
        out_specs = [_rows(H), _rows(1), _rows(1), _rows(H), _rows(1),
                     _rows(1), pl.BlockSpec((1, 1), lambda i: (0, 0))]
        out_shape = [jax.ShapeDtypeStruct((NPAD, H), _f32),
                     jax.ShapeDtypeStruct((NPAD, 1), _f32),
                     jax.ShapeDtypeStruct((NPAD, 1), _f32),
                     jax.ShapeDtypeStruct((NPAD, H), _f32),
                     jax.ShapeDtypeStruct((NPAD, 1), _i32),
                     jax.ShapeDtypeStruct((NPAD, 1), _f32),
                     jax.ShapeDtypeStruct((1, 1), _i32)]
        if has_next:
            in_specs.append(_full((H, H)))
            out_specs.append(_rows(H))
            out_shape.append(jax.ShapeDtypeStruct((NPAD, H), _f32))
        return pl.pallas_call(
            body, grid=(GRID,), in_specs=in_specs, out_specs=out_specs,
            out_shape=out_shape,
        )(aggp0, aggp1, hp_prev, dinv, b_l, x_prev, temp_prev, soft_cur,
          rs0, rs1, dcc, z, hard, exitl, gh, wt_t, ws1, bs1r, ws2_t, bs2r,
          wh1a, wh1b, bh1r, wh2_t, bh2r, gs, *wn)
    return call


def _tc_final(x3, temp3, soft3, rs0, rs1, dcc, z, hard, exitl, gh,
              wh1a, wh1b, bh1r, wh2_t, bh2r):
    def body(x_ref, tp_ref, sc_ref, r0_ref, r1_ref, dcc_ref, z_ref, hd_ref,
             ex_ref, gh_ref, wh1a_ref, wh1b_ref, bh1_ref, wh2_ref, bh2_ref,
             z_out, ex_out):
        z_new, exit_new, hard_new, _ = _hard_update(
            x_ref[...], r0_ref[...], r1_ref[...], dcc_ref[...],
            wh1a_ref[...], wh1b_ref[...], bh1_ref[...], wh2_ref[...],
            bh2_ref[...], gh_ref[...], tp_ref[...], sc_ref[...],
            z_ref[...], hd_ref[...], ex_ref[...], 2)
        z_out[...] = z_new + x_ref[...] * (1.0 - hard_new)
        ex_out[...] = exit_new
    return pl.pallas_call(
        body, grid=(GRID,),
        in_specs=[_rows(H), _rows(1), _rows(1), _rows(1), _rows(1),
                  _rows(1), _rows(H), _rows(1), _rows(1), _rows(2),
                  _full((H, C_HID)), _full((1, C_HID)), _full((1, C_HID)),
                  _full((2, C_HID)), _full((1, 2))],
        out_specs=[_rows(H), _rows(1)],
        out_shape=[jax.ShapeDtypeStruct((NPAD, H), _f32),
                   jax.ShapeDtypeStruct((NPAD, 1), _i32)],
    )(x3, temp3, soft3, rs0, rs1, dcc, z, hard, exitl, gh,
      wh1a, wh1b, bh1r, wh2_t, bh2r)


def kernel(x, edge_index, W1, b1, W2, b2, W3, b3, Wt, Ws1, bs1, Ws2, bs2,
           Wh1, bh1, Wh2, bh2):
    row = edge_index[0].astype(_i32)
    col = edge_index[1].astype(_i32)
    fill = N + (jnp.arange(EPAD - E, dtype=_i32) % (NPAD - N))
    rowp2 = jnp.concatenate([row, fill]).reshape(NW, NB, B)
    colp2 = jnp.concatenate([col, fill]).reshape(NW, NB, B)

    x_pad = jnp.pad(x, ((0, NPAD - N), (0, 0)))
    zeros1 = jnp.zeros((RPT,), _f32)
    zeros2d = jnp.zeros((B, H), _f32)

    gs = [jnp.pad(jax.random.gumbel(jax.random.fold_in(jax.random.key(1), li),
                                    (N, 2), _f32), ((0, NPAD - N), (0, 0)))
          for li in range(3)]
    gh = [jnp.pad(jax.random.gumbel(jax.random.fold_in(jax.random.key(2), li),
                                    (N, 2), _f32), ((0, NPAD - N), (0, 0)))
          for li in range(3)]

    wt_t = Wt.T
    ws2_t = Ws2.T
    wh1a = Wh1[:H, :]
    wh1b = Wh1[H:H + 1, :]
    wh2_t = Wh2.T
    b1r = b1.reshape(1, H)
    b2r = b2.reshape(1, H)
    b3r = b3.reshape(1, H)
    bs1r = bs1.reshape(1, C_HID)
    bs2r = bs2.reshape(1, 2)
    bh1r = bh1.reshape(1, C_HID)
    bh2r = bh2.reshape(1, 2)

    degp, dcp = _degdc_kernel(rowp2, colp2, zeros1)
    deg0 = degp[0].reshape(NPAD, 1)
    deg1 = degp[1].reshape(NPAD, 1)
    dc0 = dcp[0].reshape(NPAD, 1)
    dc1 = dcp[1].reshape(NPAD, 1)

    hp0, dinv, dcc = _tc_prep(x_pad, W1, deg0, deg1, dc0, dc1)

    [aggp] = _agg_kernel(hp0, rowp2, colp2, zeros2d)
    x1, temp1, soft1, hp1 = _tc_layer0(
        aggp[0], aggp[1], hp0, dinv, b1r, wt_t, Ws1, bs1r, ws2_t, bs2r,
        gs[0], W2)

    aggp, rsp = _agg_rs_kernel(hp1, rowp2, colp2, zeros2d,
                               soft1.reshape(NPAD), zeros1)
    z0 = jnp.zeros((NPAD, H), _f32)
    hard0 = jnp.zeros((NPAD, 1), _f32)
    exit0 = jnp.full((NPAD, 1), 3, _i32)
    (x2, temp2, soft2, z1, exit1, hard1, cnt0, hp2) = _tc_mid(0, True, True)(
        aggp[0], aggp[1], hp1, dinv, b2r, x1, temp1, soft1,
        rsp[0].reshape(NPAD, 1), rsp[1].reshape(NPAD, 1), dcc,
        z0, hard0, exit0, gh[0], wt_t, Ws1, bs1r, ws2_t, bs2r,
        wh1a, wh1b, bh1r, wh2_t, bh2r, gs[1], W3)

    aggp, rsp = _agg_rs_kernel(hp2, rowp2, colp2, zeros2d,
                               soft2.reshape(NPAD), zeros1)
    (x3, temp3, soft3, z2, exit2, hard2, cnt1) = _tc_mid(1, False, False)(
        aggp[0], aggp[1], hp2, dinv, b3r, x2, temp2, soft2,
        rsp[0].reshape(NPAD, 1), rsp[1].reshape(NPAD, 1), dcc,
        z1, hard1, exit1, gh[1], wt_t, Ws1, bs1r, ws2_t, bs2r,
        wh1a, wh1b, bh1r, wh2_t, bh2r, gs[2])

    [rsp] = _rs_kernel(rowp2, colp2, soft3.reshape(NPAD), zeros1)
    z3, exit3 = _tc_final(
        x3, temp3, soft3, rsp[0].reshape(NPAD, 1), rsp[1].reshape(NPAD, 1),
        dcc, z2, hard2, exit2, gh[2], wh1a, wh1b, bh1r, wh2_t, bh2r)

    c0 = cnt0[0, 0]
    c1 = cnt1[0, 0]
    n = jnp.int32(N)
    active = jnp.stack([n, n - c0, n - c0 - c1])
    return z3[:N], exit3[:N, 0], active


# speedup vs baseline: 23.0048x; 1.2679x over previous
import functools

import jax
import jax.numpy as jnp
from jax import lax
from jax.experimental import pallas as pl
from jax.experimental.pallas import tpu as pltpu
from jax.experimental.pallas import tpu_sc as plsc

N = 10000
E = 320000
H = 128
C_HID = 64
TAU0 = 1.0

NPAD = 10240
BN = 1024
GRID = NPAD // BN
NC = 2
NS = 16
NW = NC * NS
B = 64
NB = 160
PER_W = NB * B
EPAD = NW * PER_W
RS_CH = 40
RS_NCH = NB // RS_CH
RPT = NPAD // NS

_f32 = jnp.float32
_i32 = jnp.int32


def _mesh():
    return plsc.VectorSubcoreMesh(core_axis_name="c", subcore_axis_name="s")


def _make_degdc():
    @functools.partial(
        pl.kernel, mesh=_mesh(),
        out_type=[jax.ShapeDtypeStruct((NC, NPAD), _f32),
                  jax.ShapeDtypeStruct((NC, NPAD), _f32)],
        scratch_types=[
            pltpu.VMEM((NB, B), _i32),
            pltpu.VMEM((NB, B), _i32),
            pltpu.VMEM((B,), _f32),
            pltpu.VMEM((NB, B), _f32),
            pltpu.VMEM_SHARED((NPAD,), _f32),
            pltpu.VMEM_SHARED((NPAD,), _f32),
            pltpu.SemaphoreType.DMA,
        ],
    )
    def degdc(rowp2, colp2, zeros1, degp_out, dcp_out,
              row_v, col_v, ones_v, mask2_v, deg_sh, dc_sh, sem_d):
        c = lax.axis_index("c")
        s = lax.axis_index("s")
        wid = s * NC + c
        pltpu.sync_copy(zeros1, deg_sh.at[pl.ds(s * RPT, RPT)])
        pltpu.sync_copy(zeros1, dc_sh.at[pl.ds(s * RPT, RPT)])
        pltpu.sync_copy(rowp2.at[wid], row_v)
        pltpu.sync_copy(colp2.at[wid], col_v)
        for k in range(B // 16):
            ones_v[pl.ds(k * 16, 16)] = jnp.full((16,), 1.0, _f32)
        plsc.subcore_barrier()

        def fire_deg(b, carry):
            pltpu.async_copy(ones_v, deg_sh.at[col_v.at[b]], sem_d,
                             add=True)
            return carry
        lax.fori_loop(0, NB, fire_deg, 0)

        def mask_batch(b, carry):
            def lanes(k, carry2):
                rr = row_v[b, pl.ds(k * 16, 16)]
                cc = col_v[b, pl.ds(k * 16, 16)]
                mask2_v[b, pl.ds(k * 16, 16)] = jnp.where(
                    rr != cc, jnp.full((16,), 1.0, _f32),
                    jnp.zeros((16,), _f32))
                return carry2
            lax.fori_loop(0, B // 16, lanes, 0)
            pltpu.async_copy(mask2_v.at[b], dc_sh.at[row_v.at[b]], sem_d,
                             add=True)
            return carry
        lax.fori_loop(0, NB, mask_batch, 0)

        def drain_deg(b, carry):
            pltpu.make_async_copy(ones_v, deg_sh.at[col_v.at[b]],
                                  sem_d).wait()
            return carry
        lax.fori_loop(0, NB, drain_deg, 0)

        def drain_dc(b, carry):
            pltpu.make_async_copy(mask2_v.at[b], dc_sh.at[row_v.at[b]],
                                  sem_d).wait()
            return carry
        lax.fori_loop(0, NB, drain_dc, 0)

        plsc.subcore_barrier()
        pltpu.sync_copy(deg_sh.at[pl.ds(s * RPT, RPT)],
                        degp_out.at[c, pl.ds(s * RPT, RPT)])
        pltpu.sync_copy(dc_sh.at[pl.ds(s * RPT, RPT)],
                        dcp_out.at[c, pl.ds(s * RPT, RPT)])
    return degdc


def _make_edge_pass(do_agg, do_rs):
    out_type = []
    if do_agg:
        out_type.append(jax.ShapeDtypeStruct((NC, NPAD, H), _f32))
    if do_rs:
        out_type.append(jax.ShapeDtypeStruct((NC, NPAD), _f32))
    CHB = RS_CH if do_agg else NB
    NCH = NB // CHB
    scratch = [pltpu.VMEM((CHB, B), _i32), pltpu.VMEM((CHB, B), _i32)]
    if do_agg:
        scratch += [pltpu.VMEM((B, H), _f32),
                    pltpu.VMEM((B, H), _f32),
                    pltpu.VMEM_SHARED((NPAD, H), _f32),
                    pltpu.SemaphoreType.DMA,
                    pltpu.SemaphoreType.DMA]
    if do_rs:
        scratch += [pltpu.VMEM((CHB, B), _f32),
                    pltpu.VMEM((CHB, B), _f32),
                    pltpu.VMEM_SHARED((NPAD,), _f32),
                    pltpu.SemaphoreType.DMA,
                    pltpu.SemaphoreType.DMA]

    @functools.partial(pl.kernel, mesh=_mesh(), out_type=out_type,
                       scratch_types=scratch)
    def edge_pass(*refs):
        it = iter(refs)
        hp = next(it) if do_agg else None
        rowp2 = next(it)
        colp2 = next(it)
        zeros2d = next(it) if do_agg else None
        soft = next(it) if do_rs else None
        zeros1 = next(it) if do_rs else None
        aggp_out = next(it) if do_agg else None
        rsp_out = next(it) if do_rs else None
        row_v = next(it)
        col_v = next(it)
        if do_agg:
            vals_a = next(it)
            vals_b = next(it)
            agg_sh = next(it)
            sem_a = next(it)
            sem_b = next(it)
        if do_rs:
            sval2_v = next(it)
            nse2_v = next(it)
            rs_sh = next(it)
            sem_rs = next(it)
            sem_rs2 = next(it)

        c = lax.axis_index("c")
        s = lax.axis_index("s")
        wid = s * NC + c
        if do_agg:
            for j in range(RPT // B):
                pltpu.sync_copy(zeros2d,
                                agg_sh.at[pl.ds(s * RPT + j * B, B)])
        if do_rs:
            pltpu.sync_copy(zeros1, rs_sh.at[pl.ds(s * RPT, RPT)])
        plsc.subcore_barrier()

        if do_agg:
            npair = CHB // 2
            for ch in range(NCH):
                base = ch * CHB
                pltpu.sync_copy(rowp2.at[wid, pl.ds(base, CHB)], row_v)
                pltpu.sync_copy(colp2.at[wid, pl.ds(base, CHB)], col_v)
                pltpu.async_copy(hp.at[col_v.at[0]], vals_a, sem_a)

                def pair(p, carry):
                    b0 = p * 2
                    b1 = b0 + 1
                    pltpu.async_copy(hp.at[col_v.at[b1]], vals_b, sem_b)
                    pltpu.make_async_copy(hp.at[col_v.at[b0]], vals_a,
                                          sem_a).wait()
                    pltpu.sync_copy(vals_a, agg_sh.at[row_v.at[b0]],
                                    add=True)

                    @pl.when(p < npair - 1)
                    def _():
                        pltpu.async_copy(hp.at[col_v.at[b0 + 2]], vals_a,
                                         sem_a)
                    pltpu.make_async_copy(hp.at[col_v.at[b1]], vals_b,
                                          sem_b).wait()
                    pltpu.sync_copy(vals_b, agg_sh.at[row_v.at[b1]],
                                    add=True)
                    return carry
                lax.fori_loop(0, npair, pair, 0)

        if do_rs:
            for ch in range(NCH):
                base = ch * CHB
                pltpu.sync_copy(rowp2.at[wid, pl.ds(base, CHB)], row_v)
                pltpu.sync_copy(colp2.at[wid, pl.ds(base, CHB)], col_v)

                def fire(i, carry):
                    pltpu.async_copy(soft.at[col_v.at[i]], sval2_v.at[i],
                                     sem_rs)
                    return carry
                lax.fori_loop(0, CHB, fire, 0)

                def rs_batch(i, carry):
                    pltpu.make_async_copy(soft.at[col_v.at[i]],
                                          sval2_v.at[i], sem_rs).wait()

                    def lanes(k, carry2):
                        rr = row_v[i, pl.ds(k * 16, 16)]
                        cc = col_v[i, pl.ds(k * 16, 16)]
                        sv = sval2_v[i, pl.ds(k * 16, 16)]
                        nse2_v[i, pl.ds(k * 16, 16)] = jnp.where(
                            rr != cc, sv, jnp.zeros((16,), _f32))
                        return carry2
                    lax.fori_loop(0, B // 16, lanes, 0)
                    pltpu.async_copy(nse2_v.at[i], rs_sh.at[row_v.at[i]],
                                     sem_rs2, add=True)
                    return carry
                lax.fori_loop(0, CHB, rs_batch, 0)

                def rs_drain(i, carry):
                    pltpu.make_async_copy(nse2_v.at[i],
                                          rs_sh.at[row_v.at[i]],
                                          sem_rs2).wait()
                    return carry
                lax.fori_loop(0, CHB, rs_drain, 0)

        plsc.subcore_barrier()
        if do_agg:
            pltpu.sync_copy(agg_sh.at[pl.ds(s * RPT, RPT)],
                            aggp_out.at[c, pl.ds(s * RPT, RPT)])
        if do_rs:
            pltpu.sync_copy(rs_sh.at[pl.ds(s * RPT, RPT)],
                            rsp_out.at[c, pl.ds(s * RPT, RPT)])
    return edge_pass


_degdc_kernel = _make_degdc()
_agg_kernel = _make_edge_pass(True, False)
_agg_rs_kernel = _make_edge_pass(True, True)
_rs_kernel = _make_edge_pass(False, True)


def _full(shape):
    return pl.BlockSpec(shape, lambda i: tuple(0 for _ in shape))


def _rows(width):
    return pl.BlockSpec((BN, width), lambda i: (i, 0))


def _dot(a, b):
    return jnp.dot(a, b, preferred_element_type=_f32)


_SQRT_HALF = 0.7071067811865476


def _gelu(x):
    return 0.5 * x * (1.0 - lax.erf(-x * _SQRT_HALF))


def _temp_soft(xb, wt, ws1, bs1, ws2, bs2, g, soft_prev):
    val = jax.nn.softplus(_dot(xb, wt)) + TAU0
    temp = 1.0 / val
    temp = jnp.where(jnp.isinf(temp), 0.0, temp)
    a = jax.nn.relu(_dot(xb, ws1) + bs1)
    logits = _dot(a, ws2) + bs2
    aa = (logits + g) / temp
    m = jnp.max(aa, axis=-1, keepdims=True)
    e = jnp.exp(aa - m)
    p = e / jnp.sum(e, axis=-1, keepdims=True)
    sdf = (p[:, 1:2] > p[:, 0:1]).astype(_f32)
    if soft_prev is None:
        return temp, sdf
    return temp, soft_prev + sdf * (1.0 - soft_prev)


def _hard_update(xb, rs0, rs1, dcc, wh1, bh1, wh2, bh2, g,
                 temp, softb, hard, exitl, li):
    readiness = (rs0 + rs1) / dcc
    cat = jnp.concatenate([xb, readiness], axis=-1)
    a = jax.nn.relu(_dot(cat, wh1) + bh1)
    logits = _dot(a, wh2) + bh2
    cont = jnp.concatenate(
        [jnp.full((xb.shape[0], 1), 1000000.0, _f32),
         jnp.zeros((xb.shape[0], 1), _f32)], axis=1)
    eff = jnp.where(softb > 0.0, logits, cont)
    aa = (eff + g) / temp
    m = jnp.max(aa, axis=-1, keepdims=True)
    e = jnp.exp(aa - m)
    p = e / jnp.sum(e, axis=-1, keepdims=True)
    nh = (p[:, 1:2] > p[:, 0:1]).astype(_f32) * (1.0 - hard)
    exit_new = jnp.where(nh > 0.0, jnp.full_like(exitl, li), exitl)
    hard_new = hard + nh
    return exit_new, hard_new, nh


def _tc_prep(x_pad, W1, deg0, deg1, dc0, dc1):
    def body(x_ref, w_ref, d0_ref, d1_ref, c0_ref, c1_ref,
             hp_ref, dinv_ref, dcc_ref):
        deg = d0_ref[...] + d1_ref[...] + 1.0
        dinv = lax.rsqrt(deg)
        dcc_ref[...] = jnp.maximum(c0_ref[...] + c1_ref[...], 1.0)
        h = _dot(x_ref[...], w_ref[...])
        hp_ref[...] = h * dinv
        dinv_ref[...] = dinv
    return pl.pallas_call(
        body, grid=(GRID,),
        in_specs=[_rows(H), _full((H, H)), _rows(1), _rows(1), _rows(1),
                  _rows(1)],
        out_specs=[_rows(H), _rows(1), _rows(1)],
        out_shape=[jax.ShapeDtypeStruct((NPAD, H), _f32),
                   jax.ShapeDtypeStruct((NPAD, 1), _f32),
                   jax.ShapeDtypeStruct((NPAD, 1), _f32)],
    )(x_pad, W1, deg0, deg1, dc0, dc1)


def _tc_layer0(aggp0, aggp1, hp0, dinv, b1r, wt_t, ws1, bs1r, ws2_t, bs2r,
               gs0, W2):
    def body(p0_ref, p1_ref, hp_ref, dinv_ref, b_ref, wt_ref, ws1_ref,
             bs1_ref, ws2_ref, bs2_ref, g_ref, wn_ref,
             x_ref, temp_ref, soft_ref, hpn_ref):
        dinv = dinv_ref[...]
        agg = dinv * (p0_ref[...] + p1_ref[...]) + dinv * hp_ref[...] \
            + b_ref[...]
        x1 = _gelu(agg)
        temp, soft = _temp_soft(x1, wt_ref[...], ws1_ref[...], bs1_ref[...],
                                ws2_ref[...], bs2_ref[...], g_ref[...], None)
        x_ref[...] = x1
        temp_ref[...] = temp
        soft_ref[...] = soft
        hpn_ref[...] = _dot(x1, wn_ref[...]) * dinv
    return pl.pallas_call(
        body, grid=(GRID,),
        in_specs=[_rows(H), _rows(H), _rows(H), _rows(1), _full((1, H)),
                  _full((H, 1)), _full((H, C_HID)), _full((1, C_HID)),
                  _full((C_HID, 2)), _full((1, 2)), _rows(2),
                  _full((H, H))],
        out_specs=[_rows(H), _rows(1), _rows(1), _rows(H)],
        out_shape=[jax.ShapeDtypeStruct((NPAD, H), _f32),
                   jax.ShapeDtypeStruct((NPAD, 1), _f32),
                   jax.ShapeDtypeStruct((NPAD, 1), _f32),
                   jax.ShapeDtypeStruct((NPAD, H), _f32)],
    )(aggp0, aggp1, hp0, dinv, b1r, wt_t, ws1, bs1r, ws2_t, bs2r, gs0, W2)


def _tc_mid(li_hard, act, has_next):
    def call(aggp0, aggp1, hp_prev, dinv, b_l, x_prev, temp_prev, soft_cur,
             rs0, rs1, dcc, hard, exitl, gh, wt, ws1, bs1r, ws2,
             bs2r, wh1, bh1r, wh2, bh2r, gs, *wn):
        def body(*refs):
            (p0_ref, p1_ref, hp_ref, dinv_ref, b_ref, xp_ref, tp_ref,
             sc_ref, r0_ref, r1_ref, dcc_ref, hd_ref, ex_ref,
             gh_ref, wt_ref, ws1_ref, bs1_ref, ws2_ref, bs2_ref, wh1_ref,
             bh1_ref, wh2_ref, bh2_ref, gs_ref) = refs[:24]
            idx = 24
            wn_ref = refs[idx] if has_next else None
            idx += 1 if has_next else 0
            outs = refs[idx:]
            (x_ref, temp_ref, soft_ref, ex_out, hd_out,
             cnt_ref) = outs[:6]
            hpn_ref = outs[6] if has_next else None

            i = pl.program_id(0)
            exit_new, hard_new, nh = _hard_update(
                xp_ref[...], r0_ref[...], r1_ref[...], dcc_ref[...],
                wh1_ref[...], bh1_ref[...], wh2_ref[...],
                bh2_ref[...], gh_ref[...], tp_ref[...], sc_ref[...],
                hd_ref[...], ex_ref[...], li_hard)
            ex_out[...] = exit_new
            hd_out[...] = hard_new
            ids = lax.broadcasted_iota(_i32, (BN, 1), 0) + i * BN
            cnt = jnp.sum(jnp.where(ids < N, nh, 0.0)).astype(_i32)
            prev = cnt_ref[...]
            base = jnp.where(i == 0, jnp.zeros_like(prev), prev)
            cnt_ref[...] = base + cnt[None, None]

            dinv = dinv_ref[...]
            agg = dinv * (p0_ref[...] + p1_ref[...]) + dinv * hp_ref[...] \
                + b_ref[...]
            x_new = _gelu(agg) if act else agg
            temp, soft = _temp_soft(
                x_new, wt_ref[...], ws1_ref[...], bs1_ref[...], ws2_ref[...],
                bs2_ref[...], gs_ref[...], sc_ref[...])
            x_ref[...] = x_new
            temp_ref[...] = temp
            soft_ref[...] = soft
            if has_next:
                hpn_ref[...] = _dot(x_new, wn_ref[...]) * dinv

        in_specs = [_rows(H), _rows(H), _rows(H), _rows(1), _full((1, H)),
                    _rows(H), _rows(1), _rows(1), _rows(1), _rows(1),
                    _rows(1), _rows(1), _rows(1), _rows(2),
                    _full((H, 1)), _full((H, C_HID)), _full((1, C_HID)),
                    _full((C_HID, 2)), _full((1, 2)),
                    _full((H + 1, C_HID)), _full((1, C_HID)),
                    _full((C_HID, 2)), _full((1, 2)), _rows(2)]
        out_specs = [_rows(H), _rows(1), _rows(1), _rows(1),
                     _rows(1), pl.BlockSpec((1, 1), lambda i: (0, 0))]
        out_shape = [jax.ShapeDtypeStruct((NPAD, H), _f32),
                     jax.ShapeDtypeStruct((NPAD, 1), _f32),
                     jax.ShapeDtypeStruct((NPAD, 1), _f32),
                     jax.ShapeDtypeStruct((NPAD, 1), _i32),
                     jax.ShapeDtypeStruct((NPAD, 1), _f32),
                     jax.ShapeDtypeStruct((1, 1), _i32)]
        if has_next:
            in_specs.append(_full((H, H)))
            out_specs.append(_rows(H))
            out_shape.append(jax.ShapeDtypeStruct((NPAD, H), _f32))
        return pl.pallas_call(
            body, grid=(GRID,), in_specs=in_specs, out_specs=out_specs,
            out_shape=out_shape,
        )(aggp0, aggp1, hp_prev, dinv, b_l, x_prev, temp_prev, soft_cur,
          rs0, rs1, dcc, hard, exitl, gh, wt, ws1, bs1r, ws2, bs2r,
          wh1, bh1r, wh2, bh2r, gs, *wn)
    return call


def _tc_final(x1, x2, x3, temp3, soft3, rs0, rs1, dcc, hard, exitl, gh,
              wh1, bh1r, wh2, bh2r):
    def body(x1_ref, x2_ref, x3_ref, tp_ref, sc_ref, r0_ref, r1_ref,
             dcc_ref, hd_ref, ex_ref, gh_ref, wh1_ref, bh1_ref,
             wh2_ref, bh2_ref, z_out, ex_out):
        exit_new, _, _ = _hard_update(
            x3_ref[...], r0_ref[...], r1_ref[...], dcc_ref[...],
            wh1_ref[...], bh1_ref[...], wh2_ref[...],
            bh2_ref[...], gh_ref[...], tp_ref[...], sc_ref[...],
            hd_ref[...], ex_ref[...], 2)
        z_out[...] = jnp.where(exit_new == 0, x1_ref[...],
                               jnp.where(exit_new == 1, x2_ref[...],
                                         x3_ref[...]))
        ex_out[...] = exit_new
    return pl.pallas_call(
        body, grid=(GRID,),
        in_specs=[_rows(H), _rows(H), _rows(H), _rows(1), _rows(1),
                  _rows(1), _rows(1), _rows(1), _rows(1), _rows(1),
                  _rows(2),
                  _full((H + 1, C_HID)), _full((1, C_HID)),
                  _full((C_HID, 2)), _full((1, 2))],
        out_specs=[_rows(H), _rows(1)],
        out_shape=[jax.ShapeDtypeStruct((NPAD, H), _f32),
                   jax.ShapeDtypeStruct((NPAD, 1), _i32)],
    )(x1, x2, x3, temp3, soft3, rs0, rs1, dcc, hard, exitl, gh,
      wh1, bh1r, wh2, bh2r)


def kernel(x, edge_index, W1, b1, W2, b2, W3, b3, Wt, Ws1, bs1, Ws2, bs2,
           Wh1, bh1, Wh2, bh2):
    row = edge_index[0].astype(_i32)
    col = edge_index[1].astype(_i32)
    fill = N + (jnp.arange(EPAD - E, dtype=_i32) % (NPAD - N))
    rowp2 = jnp.concatenate([row, fill]).reshape(NW, NB, B)
    colp2 = jnp.concatenate([col, fill]).reshape(NW, NB, B)

    x_pad = jnp.pad(x, ((0, NPAD - N), (0, 0)))
    zeros1 = jnp.zeros((RPT,), _f32)
    zeros2d = jnp.zeros((B, H), _f32)

    gs = [jnp.pad(jax.random.gumbel(jax.random.fold_in(jax.random.key(1), li),
                                    (N, 2), _f32), ((0, NPAD - N), (0, 0)))
          for li in range(3)]
    gh = [jnp.pad(jax.random.gumbel(jax.random.fold_in(jax.random.key(2), li),
                                    (N, 2), _f32), ((0, NPAD - N), (0, 0)))
          for li in range(3)]

    b1r = b1.reshape(1, H)
    b2r = b2.reshape(1, H)
    b3r = b3.reshape(1, H)
    bs1r = bs1.reshape(1, C_HID)
    bs2r = bs2.reshape(1, 2)
    bh1r = bh1.reshape(1, C_HID)
    bh2r = bh2.reshape(1, 2)

    degp, dcp = _degdc_kernel(rowp2, colp2, zeros1)
    deg0 = degp[0].reshape(NPAD, 1)
    deg1 = degp[1].reshape(NPAD, 1)
    dc0 = dcp[0].reshape(NPAD, 1)
    dc1 = dcp[1].reshape(NPAD, 1)

    hp0, dinv, dcc = _tc_prep(x_pad, W1, deg0, deg1, dc0, dc1)

    [aggp] = _agg_kernel(hp0, rowp2, colp2, zeros2d)
    x1, temp1, soft1, hp1 = _tc_layer0(
        aggp[0], aggp[1], hp0, dinv, b1r, Wt, Ws1, bs1r, Ws2, bs2r,
        gs[0], W2)

    aggp, rsp = _agg_rs_kernel(hp1, rowp2, colp2, zeros2d,
                               soft1.reshape(NPAD), zeros1)
    hard0 = jnp.zeros((NPAD, 1), _f32)
    exit0 = jnp.full((NPAD, 1), 3, _i32)
    (x2, temp2, soft2, exit1, hard1, cnt0, hp2) = _tc_mid(0, True, True)(
        aggp[0], aggp[1], hp1, dinv, b2r, x1, temp1, soft1,
        rsp[0].reshape(NPAD, 1), rsp[1].reshape(NPAD, 1), dcc,
        hard0, exit0, gh[0], Wt, Ws1, bs1r, Ws2, bs2r,
        Wh1, bh1r, Wh2, bh2r, gs[1], W3)

    aggp, rsp = _agg_rs_kernel(hp2, rowp2, colp2, zeros2d,
                               soft2.reshape(NPAD), zeros1)
    (x3, temp3, soft3, exit2, hard2, cnt1) = _tc_mid(1, False, False)(
        aggp[0], aggp[1], hp2, dinv, b3r, x2, temp2, soft2,
        rsp[0].reshape(NPAD, 1), rsp[1].reshape(NPAD, 1), dcc,
        hard1, exit1, gh[1], Wt, Ws1, bs1r, Ws2, bs2r,
        Wh1, bh1r, Wh2, bh2r, gs[2])

    [rsp] = _rs_kernel(rowp2, colp2, soft3.reshape(NPAD), zeros1)
    z3, exit3 = _tc_final(
        x1, x2, x3, temp3, soft3,
        rsp[0].reshape(NPAD, 1), rsp[1].reshape(NPAD, 1),
        dcc, hard2, exit2, gh[2], Wh1, bh1r, Wh2, bh2r)

    c0 = cnt0[0, 0]
    c1 = cnt1[0, 0]
    n = jnp.int32(N)
    active = jnp.stack([n, n - c0, n - c0 - c1])
    return z3[:N], exit3[:N, 0], active


# speedup vs baseline: 23.4528x; 1.0195x over previous
import functools

import jax
import jax.numpy as jnp
from jax import lax
from jax.experimental import pallas as pl
from jax.experimental.pallas import tpu as pltpu
from jax.experimental.pallas import tpu_sc as plsc

N = 10000
E = 320000
H = 128
C_HID = 64
TAU0 = 1.0

NPAD = 10240
BN = 1024
GRID = NPAD // BN
NC = 2
NS = 16
NW = NC * NS
B = 64
NB = 160
PER_W = NB * B
EPAD = NW * PER_W
RS_CH = 40
RS_NCH = NB // RS_CH
RPT = NPAD // NS

_f32 = jnp.float32
_i32 = jnp.int32


def _mesh():
    return plsc.VectorSubcoreMesh(core_axis_name="c", subcore_axis_name="s")


def _make_degdc():
    @functools.partial(
        pl.kernel, mesh=_mesh(),
        out_type=[jax.ShapeDtypeStruct((NC, NPAD), _f32),
                  jax.ShapeDtypeStruct((NC, NPAD), _f32)],
        scratch_types=[
            pltpu.VMEM((NB, B), _i32),
            pltpu.VMEM((NB, B), _i32),
            pltpu.VMEM((B,), _f32),
            pltpu.VMEM((NB, B), _f32),
            pltpu.VMEM_SHARED((NPAD,), _f32),
            pltpu.VMEM_SHARED((NPAD,), _f32),
            pltpu.SemaphoreType.DMA,
        ],
    )
    def degdc(rowp2, colp2, zeros1, degp_out, dcp_out,
              row_v, col_v, ones_v, mask2_v, deg_sh, dc_sh, sem_d):
        c = lax.axis_index("c")
        s = lax.axis_index("s")
        wid = s * NC + c
        pltpu.sync_copy(zeros1, deg_sh.at[pl.ds(s * RPT, RPT)])
        pltpu.sync_copy(zeros1, dc_sh.at[pl.ds(s * RPT, RPT)])
        pltpu.sync_copy(rowp2.at[wid], row_v)
        pltpu.sync_copy(colp2.at[wid], col_v)
        for k in range(B // 16):
            ones_v[pl.ds(k * 16, 16)] = jnp.full((16,), 1.0, _f32)
        plsc.subcore_barrier()

        def fire_deg(b, carry):
            pltpu.async_copy(ones_v, deg_sh.at[col_v.at[b]], sem_d,
                             add=True)
            return carry
        lax.fori_loop(0, NB, fire_deg, 0)

        def mask_batch(b, carry):
            def lanes(k, carry2):
                rr = row_v[b, pl.ds(k * 16, 16)]
                cc = col_v[b, pl.ds(k * 16, 16)]
                mask2_v[b, pl.ds(k * 16, 16)] = jnp.where(
                    rr != cc, jnp.full((16,), 1.0, _f32),
                    jnp.zeros((16,), _f32))
                return carry2
            lax.fori_loop(0, B // 16, lanes, 0)
            pltpu.async_copy(mask2_v.at[b], dc_sh.at[row_v.at[b]], sem_d,
                             add=True)
            return carry
        lax.fori_loop(0, NB, mask_batch, 0)

        def drain_deg(b, carry):
            pltpu.make_async_copy(ones_v, deg_sh.at[col_v.at[b]],
                                  sem_d).wait()
            return carry
        lax.fori_loop(0, NB, drain_deg, 0)

        def drain_dc(b, carry):
            pltpu.make_async_copy(mask2_v.at[b], dc_sh.at[row_v.at[b]],
                                  sem_d).wait()
            return carry
        lax.fori_loop(0, NB, drain_dc, 0)

        plsc.subcore_barrier()
        pltpu.sync_copy(deg_sh.at[pl.ds(s * RPT, RPT)],
                        degp_out.at[c, pl.ds(s * RPT, RPT)])
        pltpu.sync_copy(dc_sh.at[pl.ds(s * RPT, RPT)],
                        dcp_out.at[c, pl.ds(s * RPT, RPT)])
    return degdc


def _make_edge_pass(do_agg, do_rs):
    out_type = []
    if do_agg:
        out_type.append(jax.ShapeDtypeStruct((NC, NPAD, H), _f32))
    if do_rs:
        out_type.append(jax.ShapeDtypeStruct((NC, NPAD), _f32))
    CHB = RS_CH if do_agg else NB
    NCH = NB // CHB
    scratch = [pltpu.VMEM((CHB, B), _i32), pltpu.VMEM((CHB, B), _i32)]
    if do_agg:
        scratch += [pltpu.VMEM((B, H), _f32),
                    pltpu.VMEM((B, H), _f32),
                    pltpu.VMEM_SHARED((NPAD, H), _f32),
                    pltpu.SemaphoreType.DMA,
                    pltpu.SemaphoreType.DMA]
    if do_rs:
        scratch += [pltpu.VMEM((CHB, B), _f32),
                    pltpu.VMEM((CHB, B), _f32),
                    pltpu.VMEM_SHARED((NPAD,), _f32),
                    pltpu.SemaphoreType.DMA,
                    pltpu.SemaphoreType.DMA]

    @functools.partial(pl.kernel, mesh=_mesh(), out_type=out_type,
                       scratch_types=scratch)
    def edge_pass(*refs):
        it = iter(refs)
        hp = next(it) if do_agg else None
        rowp2 = next(it)
        colp2 = next(it)
        zeros2d = next(it) if do_agg else None
        soft = next(it) if do_rs else None
        zeros1 = next(it) if do_rs else None
        aggp_out = next(it) if do_agg else None
        rsp_out = next(it) if do_rs else None
        row_v = next(it)
        col_v = next(it)
        if do_agg:
            vals_a = next(it)
            vals_b = next(it)
            agg_sh = next(it)
            sem_a = next(it)
            sem_b = next(it)
        if do_rs:
            sval2_v = next(it)
            nse2_v = next(it)
            rs_sh = next(it)
            sem_rs = next(it)
            sem_rs2 = next(it)

        c = lax.axis_index("c")
        s = lax.axis_index("s")
        wid = s * NC + c
        if do_agg:
            for j in range(RPT // B):
                pltpu.sync_copy(zeros2d,
                                agg_sh.at[pl.ds(s * RPT + j * B, B)])
        if do_rs:
            pltpu.sync_copy(zeros1, rs_sh.at[pl.ds(s * RPT, RPT)])
        plsc.subcore_barrier()

        def _rs_gather_fire(i, carry):
            pltpu.async_copy(soft.at[col_v.at[i]], sval2_v.at[i], sem_rs)
            return carry

        def _rs_process(i):
            pltpu.make_async_copy(soft.at[col_v.at[i]],
                                  sval2_v.at[i], sem_rs).wait()

            def lanes(k, carry2):
                rr = row_v[i, pl.ds(k * 16, 16)]
                cc = col_v[i, pl.ds(k * 16, 16)]
                sv = sval2_v[i, pl.ds(k * 16, 16)]
                nse2_v[i, pl.ds(k * 16, 16)] = jnp.where(
                    rr != cc, sv, jnp.zeros((16,), _f32))
                return carry2
            lax.fori_loop(0, B // 16, lanes, 0)
            pltpu.async_copy(nse2_v.at[i], rs_sh.at[row_v.at[i]],
                             sem_rs2, add=True)

        def _rs_drain(i, carry):
            pltpu.make_async_copy(nse2_v.at[i], rs_sh.at[row_v.at[i]],
                                  sem_rs2).wait()
            return carry

        if do_agg:
            npair = CHB // 2
            for ch in range(NCH):
                base = ch * CHB
                pltpu.sync_copy(rowp2.at[wid, pl.ds(base, CHB)], row_v)
                pltpu.sync_copy(colp2.at[wid, pl.ds(base, CHB)], col_v)
                if do_rs:
                    lax.fori_loop(0, CHB, _rs_gather_fire, 0)
                pltpu.async_copy(hp.at[col_v.at[0]], vals_a, sem_a)

                def pair(p, carry):
                    b0 = p * 2
                    b1 = b0 + 1
                    pltpu.async_copy(hp.at[col_v.at[b1]], vals_b, sem_b)
                    pltpu.make_async_copy(hp.at[col_v.at[b0]], vals_a,
                                          sem_a).wait()
                    pltpu.sync_copy(vals_a, agg_sh.at[row_v.at[b0]],
                                    add=True)

                    @pl.when(p < npair - 1)
                    def _():
                        pltpu.async_copy(hp.at[col_v.at[b0 + 2]], vals_a,
                                         sem_a)
                    if do_rs:
                        _rs_process(b0)
                    pltpu.make_async_copy(hp.at[col_v.at[b1]], vals_b,
                                          sem_b).wait()
                    pltpu.sync_copy(vals_b, agg_sh.at[row_v.at[b1]],
                                    add=True)
                    if do_rs:
                        _rs_process(b1)
                    return carry
                lax.fori_loop(0, npair, pair, 0)

                if do_rs:
                    lax.fori_loop(0, CHB, _rs_drain, 0)

        elif do_rs:
            for ch in range(NCH):
                base = ch * CHB
                pltpu.sync_copy(rowp2.at[wid, pl.ds(base, CHB)], row_v)
                pltpu.sync_copy(colp2.at[wid, pl.ds(base, CHB)], col_v)
                lax.fori_loop(0, CHB, _rs_gather_fire, 0)

                def rs_batch(i, carry):
                    _rs_process(i)
                    return carry
                lax.fori_loop(0, CHB, rs_batch, 0)
                lax.fori_loop(0, CHB, _rs_drain, 0)

        plsc.subcore_barrier()
        if do_agg:
            pltpu.sync_copy(agg_sh.at[pl.ds(s * RPT, RPT)],
                            aggp_out.at[c, pl.ds(s * RPT, RPT)])
        if do_rs:
            pltpu.sync_copy(rs_sh.at[pl.ds(s * RPT, RPT)],
                            rsp_out.at[c, pl.ds(s * RPT, RPT)])
    return edge_pass


_degdc_kernel = _make_degdc()
_agg_kernel = _make_edge_pass(True, False)
_agg_rs_kernel = _make_edge_pass(True, True)
_rs_kernel = _make_edge_pass(False, True)


def _full(shape):
    return pl.BlockSpec(shape, lambda i: tuple(0 for _ in shape))


def _rows(width):
    return pl.BlockSpec((BN, width), lambda i: (i, 0))


def _dot(a, b):
    return jnp.dot(a, b, preferred_element_type=_f32)


_SQRT_HALF = 0.7071067811865476


def _gelu(x):
    return 0.5 * x * (1.0 - lax.erf(-x * _SQRT_HALF))


def _temp_soft(xb, wt, ws1, bs1, ws2, bs2, g, soft_prev):
    val = jax.nn.softplus(_dot(xb, wt)) + TAU0
    temp = 1.0 / val
    temp = jnp.where(jnp.isinf(temp), 0.0, temp)
    a = jax.nn.relu(_dot(xb, ws1) + bs1)
    logits = _dot(a, ws2) + bs2
    aa = (logits + g) / temp
    m = jnp.max(aa, axis=-1, keepdims=True)
    e = jnp.exp(aa - m)
    p = e / jnp.sum(e, axis=-1, keepdims=True)
    sdf = (p[:, 1:2] > p[:, 0:1]).astype(_f32)
    if soft_prev is None:
        return temp, sdf
    return temp, soft_prev + sdf * (1.0 - soft_prev)


def _hard_update(xb, rs0, rs1, dcc, wh1, bh1, wh2, bh2, g,
                 temp, softb, hard, exitl, li):
    readiness = (rs0 + rs1) / dcc
    cat = jnp.concatenate([xb, readiness], axis=-1)
    a = jax.nn.relu(_dot(cat, wh1) + bh1)
    logits = _dot(a, wh2) + bh2
    cont = jnp.concatenate(
        [jnp.full((xb.shape[0], 1), 1000000.0, _f32),
         jnp.zeros((xb.shape[0], 1), _f32)], axis=1)
    eff = jnp.where(softb > 0.0, logits, cont)
    aa = (eff + g) / temp
    m = jnp.max(aa, axis=-1, keepdims=True)
    e = jnp.exp(aa - m)
    p = e / jnp.sum(e, axis=-1, keepdims=True)
    nh = (p[:, 1:2] > p[:, 0:1]).astype(_f32) * (1.0 - hard)
    exit_new = jnp.where(nh > 0.0, jnp.full_like(exitl, li), exitl)
    hard_new = hard + nh
    return exit_new, hard_new, nh


def _tc_prep(x_pad, W1, deg0, deg1, dc0, dc1):
    def body(x_ref, w_ref, d0_ref, d1_ref, c0_ref, c1_ref,
             hp_ref, dinv_ref, dcc_ref):
        deg = d0_ref[...] + d1_ref[...] + 1.0
        dinv = lax.rsqrt(deg)
        dcc_ref[...] = jnp.maximum(c0_ref[...] + c1_ref[...], 1.0)
        h = _dot(x_ref[...], w_ref[...])
        hp_ref[...] = h * dinv
        dinv_ref[...] = dinv
    return pl.pallas_call(
        body, grid=(GRID,),
        in_specs=[_rows(H), _full((H, H)), _rows(1), _rows(1), _rows(1),
                  _rows(1)],
        out_specs=[_rows(H), _rows(1), _rows(1)],
        out_shape=[jax.ShapeDtypeStruct((NPAD, H), _f32),
                   jax.ShapeDtypeStruct((NPAD, 1), _f32),
                   jax.ShapeDtypeStruct((NPAD, 1), _f32)],
    )(x_pad, W1, deg0, deg1, dc0, dc1)


def _tc_layer0(aggp0, aggp1, hp0, dinv, b1r, wt_t, ws1, bs1r, ws2_t, bs2r,
               gs0, W2):
    def body(p0_ref, p1_ref, hp_ref, dinv_ref, b_ref, wt_ref, ws1_ref,
             bs1_ref, ws2_ref, bs2_ref, g_ref, wn_ref,
             x_ref, temp_ref, soft_ref, hpn_ref):
        dinv = dinv_ref[...]
        agg = dinv * (p0_ref[...] + p1_ref[...]) + dinv * hp_ref[...] \
            + b_ref[...]
        x1 = _gelu(agg)
        temp, soft = _temp_soft(x1, wt_ref[...], ws1_ref[...], bs1_ref[...],
                                ws2_ref[...], bs2_ref[...], g_ref[...], None)
        x_ref[...] = x1
        temp_ref[...] = temp
        soft_ref[...] = soft
        hpn_ref[...] = _dot(x1, wn_ref[...]) * dinv
    return pl.pallas_call(
        body, grid=(GRID,),
        in_specs=[_rows(H), _rows(H), _rows(H), _rows(1), _full((1, H)),
                  _full((H, 1)), _full((H, C_HID)), _full((1, C_HID)),
                  _full((C_HID, 2)), _full((1, 2)), _rows(2),
                  _full((H, H))],
        out_specs=[_rows(H), _rows(1), _rows(1), _rows(H)],
        out_shape=[jax.ShapeDtypeStruct((NPAD, H), _f32),
                   jax.ShapeDtypeStruct((NPAD, 1), _f32),
                   jax.ShapeDtypeStruct((NPAD, 1), _f32),
                   jax.ShapeDtypeStruct((NPAD, H), _f32)],
    )(aggp0, aggp1, hp0, dinv, b1r, wt_t, ws1, bs1r, ws2_t, bs2r, gs0, W2)


def _tc_mid(li_hard, act, has_next):
    def call(aggp0, aggp1, hp_prev, dinv, b_l, x_prev, temp_prev, soft_cur,
             rs0, rs1, dcc, hard, exitl, gh, wt, ws1, bs1r, ws2,
             bs2r, wh1, bh1r, wh2, bh2r, gs, *wn):
        def body(*refs):
            (p0_ref, p1_ref, hp_ref, dinv_ref, b_ref, xp_ref, tp_ref,
             sc_ref, r0_ref, r1_ref, dcc_ref, hd_ref, ex_ref,
             gh_ref, wt_ref, ws1_ref, bs1_ref, ws2_ref, bs2_ref, wh1_ref,
             bh1_ref, wh2_ref, bh2_ref, gs_ref) = refs[:24]
            idx = 24
            wn_ref = refs[idx] if has_next else None
            idx += 1 if has_next else 0
            outs = refs[idx:]
            (x_ref, temp_ref, soft_ref, ex_out, hd_out,
             cnt_ref) = outs[:6]
            hpn_ref = outs[6] if has_next else None

            i = pl.program_id(0)
            exit_new, hard_new, nh = _hard_update(
                xp_ref[...], r0_ref[...], r1_ref[...], dcc_ref[...],
                wh1_ref[...], bh1_ref[...], wh2_ref[...],
                bh2_ref[...], gh_ref[...], tp_ref[...], sc_ref[...],
                hd_ref[...], ex_ref[...], li_hard)
            ex_out[...] = exit_new
            hd_out[...] = hard_new
            ids = lax.broadcasted_iota(_i32, (BN, 1), 0) + i * BN
            cnt = jnp.sum(jnp.where(ids < N, nh, 0.0)).astype(_i32)
            prev = cnt_ref[...]
            base = jnp.where(i == 0, jnp.zeros_like(prev), prev)
            cnt_ref[...] = base + cnt[None, None]

            dinv = dinv_ref[...]
            agg = dinv * (p0_ref[...] + p1_ref[...]) + dinv * hp_ref[...] \
                + b_ref[...]
            x_new = _gelu(agg) if act else agg
            temp, soft = _temp_soft(
                x_new, wt_ref[...], ws1_ref[...], bs1_ref[...], ws2_ref[...],
                bs2_ref[...], gs_ref[...], sc_ref[...])
            x_ref[...] = x_new
            temp_ref[...] = temp
            soft_ref[...] = soft
            if has_next:
                hpn_ref[...] = _dot(x_new, wn_ref[...]) * dinv

        in_specs = [_rows(H), _rows(H), _rows(H), _rows(1), _full((1, H)),
                    _rows(H), _rows(1), _rows(1), _rows(1), _rows(1),
                    _rows(1), _rows(1), _rows(1), _rows(2),
                    _full((H, 1)), _full((H, C_HID)), _full((1, C_HID)),
                    _full((C_HID, 2)), _full((1, 2)),
                    _full((H + 1, C_HID)), _full((1, C_HID)),
                    _full((C_HID, 2)), _full((1, 2)), _rows(2)]
        out_specs = [_rows(H), _rows(1), _rows(1), _rows(1),
                     _rows(1), pl.BlockSpec((1, 1), lambda i: (0, 0))]
        out_shape = [jax.ShapeDtypeStruct((NPAD, H), _f32),
                     jax.ShapeDtypeStruct((NPAD, 1), _f32),
                     jax.ShapeDtypeStruct((NPAD, 1), _f32),
                     jax.ShapeDtypeStruct((NPAD, 1), _i32),
                     jax.ShapeDtypeStruct((NPAD, 1), _f32),
                     jax.ShapeDtypeStruct((1, 1), _i32)]
        if has_next:
            in_specs.append(_full((H, H)))
            out_specs.append(_rows(H))
            out_shape.append(jax.ShapeDtypeStruct((NPAD, H), _f32))
        return pl.pallas_call(
            body, grid=(GRID,), in_specs=in_specs, out_specs=out_specs,
            out_shape=out_shape,
        )(aggp0, aggp1, hp_prev, dinv, b_l, x_prev, temp_prev, soft_cur,
          rs0, rs1, dcc, hard, exitl, gh, wt, ws1, bs1r, ws2, bs2r,
          wh1, bh1r, wh2, bh2r, gs, *wn)
    return call


def _tc_final(x1, x2, x3, temp3, soft3, rs0, rs1, dcc, hard, exitl, gh,
              wh1, bh1r, wh2, bh2r):
    def body(x1_ref, x2_ref, x3_ref, tp_ref, sc_ref, r0_ref, r1_ref,
             dcc_ref, hd_ref, ex_ref, gh_ref, wh1_ref, bh1_ref,
             wh2_ref, bh2_ref, z_out, ex_out):
        exit_new, _, _ = _hard_update(
            x3_ref[...], r0_ref[...], r1_ref[...], dcc_ref[...],
            wh1_ref[...], bh1_ref[...], wh2_ref[...],
            bh2_ref[...], gh_ref[...], tp_ref[...], sc_ref[...],
            hd_ref[...], ex_ref[...], 2)
        z_out[...] = jnp.where(exit_new == 0, x1_ref[...],
                               jnp.where(exit_new == 1, x2_ref[...],
                                         x3_ref[...]))
        ex_out[...] = exit_new
    return pl.pallas_call(
        body, grid=(GRID,),
        in_specs=[_rows(H), _rows(H), _rows(H), _rows(1), _rows(1),
                  _rows(1), _rows(1), _rows(1), _rows(1), _rows(1),
                  _rows(2),
                  _full((H + 1, C_HID)), _full((1, C_HID)),
                  _full((C_HID, 2)), _full((1, 2))],
        out_specs=[_rows(H), _rows(1)],
        out_shape=[jax.ShapeDtypeStruct((NPAD, H), _f32),
                   jax.ShapeDtypeStruct((NPAD, 1), _i32)],
    )(x1, x2, x3, temp3, soft3, rs0, rs1, dcc, hard, exitl, gh,
      wh1, bh1r, wh2, bh2r)


def kernel(x, edge_index, W1, b1, W2, b2, W3, b3, Wt, Ws1, bs1, Ws2, bs2,
           Wh1, bh1, Wh2, bh2):
    row = edge_index[0].astype(_i32)
    col = edge_index[1].astype(_i32)
    fill = N + (jnp.arange(EPAD - E, dtype=_i32) % (NPAD - N))
    rowp2 = jnp.concatenate([row, fill]).reshape(NW, NB, B)
    colp2 = jnp.concatenate([col, fill]).reshape(NW, NB, B)

    x_pad = jnp.pad(x, ((0, NPAD - N), (0, 0)))
    zeros1 = jnp.zeros((RPT,), _f32)
    zeros2d = jnp.zeros((B, H), _f32)

    gs = [jnp.pad(jax.random.gumbel(jax.random.fold_in(jax.random.key(1), li),
                                    (N, 2), _f32), ((0, NPAD - N), (0, 0)))
          for li in range(3)]
    gh = [jnp.pad(jax.random.gumbel(jax.random.fold_in(jax.random.key(2), li),
                                    (N, 2), _f32), ((0, NPAD - N), (0, 0)))
          for li in range(3)]

    b1r = b1.reshape(1, H)
    b2r = b2.reshape(1, H)
    b3r = b3.reshape(1, H)
    bs1r = bs1.reshape(1, C_HID)
    bs2r = bs2.reshape(1, 2)
    bh1r = bh1.reshape(1, C_HID)
    bh2r = bh2.reshape(1, 2)

    degp, dcp = _degdc_kernel(rowp2, colp2, zeros1)
    deg0 = degp[0].reshape(NPAD, 1)
    deg1 = degp[1].reshape(NPAD, 1)
    dc0 = dcp[0].reshape(NPAD, 1)
    dc1 = dcp[1].reshape(NPAD, 1)

    hp0, dinv, dcc = _tc_prep(x_pad, W1, deg0, deg1, dc0, dc1)

    [aggp] = _agg_kernel(hp0, rowp2, colp2, zeros2d)
    x1, temp1, soft1, hp1 = _tc_layer0(
        aggp[0], aggp[1], hp0, dinv, b1r, Wt, Ws1, bs1r, Ws2, bs2r,
        gs[0], W2)

    aggp, rsp = _agg_rs_kernel(hp1, rowp2, colp2, zeros2d,
                               soft1.reshape(NPAD), zeros1)
    hard0 = jnp.zeros((NPAD, 1), _f32)
    exit0 = jnp.full((NPAD, 1), 3, _i32)
    (x2, temp2, soft2, exit1, hard1, cnt0, hp2) = _tc_mid(0, True, True)(
        aggp[0], aggp[1], hp1, dinv, b2r, x1, temp1, soft1,
        rsp[0].reshape(NPAD, 1), rsp[1].reshape(NPAD, 1), dcc,
        hard0, exit0, gh[0], Wt, Ws1, bs1r, Ws2, bs2r,
        Wh1, bh1r, Wh2, bh2r, gs[1], W3)

    aggp, rsp = _agg_rs_kernel(hp2, rowp2, colp2, zeros2d,
                               soft2.reshape(NPAD), zeros1)
    (x3, temp3, soft3, exit2, hard2, cnt1) = _tc_mid(1, False, False)(
        aggp[0], aggp[1], hp2, dinv, b3r, x2, temp2, soft2,
        rsp[0].reshape(NPAD, 1), rsp[1].reshape(NPAD, 1), dcc,
        hard1, exit1, gh[1], Wt, Ws1, bs1r, Ws2, bs2r,
        Wh1, bh1r, Wh2, bh2r, gs[2])

    [rsp] = _rs_kernel(rowp2, colp2, soft3.reshape(NPAD), zeros1)
    z3, exit3 = _tc_final(
        x1, x2, x3, temp3, soft3,
        rsp[0].reshape(NPAD, 1), rsp[1].reshape(NPAD, 1),
        dcc, hard2, exit2, gh[2], Wh1, bh1r, Wh2, bh2r)

    c0 = cnt0[0, 0]
    c1 = cnt1[0, 0]
    n = jnp.int32(N)
    active = jnp.stack([n, n - c0, n - c0 - c1])
    return z3[:N], exit3[:N, 0], active
